# Initial kernel scaffold; baseline (speedup 1.0000x reference)
import jax
import jax.numpy as jnp
from jax.experimental import pallas as pl


def kernel(edge_index, pairs, emb, W1, b1, W2, b2, mW1, mb1, mW2, mb2):
    raise NotImplementedError("write your pallas kernel here")


# speedup vs baseline: 11.5643x; 11.5643x over previous
import functools

import jax
import jax.numpy as jnp
from jax import lax
from jax.experimental import pallas as pl
from jax.experimental.pallas import tpu as pltpu
from jax.experimental.pallas import tpu_sc as plsc

N = 50000
D = 64
E = 800000
P = 65536

NC = 2
NS = 16
NW = NC * NS
L = 16

CK = 512
EW = 25600
EP = NW * EW
NCHUNK = EW // CK
NP = 50176
SLAB = NP // NS
IR = CK // 128

_mesh = plsc.VectorSubcoreMesh(core_axis_name="c", subcore_axis_name="s")


def _wid():
    return lax.axis_index("c") * NS + lax.axis_index("s")


@functools.partial(
    pl.kernel,
    out_type=jax.ShapeDtypeStruct((NC * NP,), jnp.float32),
    mesh=_mesh,
    compiler_params=pltpu.CompilerParams(use_tc_tiling_on_sc=False),
    scratch_types=[
        pltpu.VMEM((IR, 128), jnp.int32),
        pltpu.VMEM((128,), jnp.float32),
        pltpu.VMEM((SLAB,), jnp.float32),
        pltpu.VMEM_SHARED((NP,), jnp.float32),
        pltpu.SemaphoreType.DMA,
        pltpu.SemaphoreType.DMA,
    ],
)
def _deg_kernel(dst2d_hbm, out_hbm, didx_v, ones_v, buf_v, acc_sh, ssem, wsem):
    cid = lax.axis_index("c")
    sid = lax.axis_index("s")
    wid = cid * NS + sid

    zero16 = jnp.zeros((L,), jnp.float32)
    one16 = jnp.ones((L,), jnp.float32)
    for q in range(128 // L):
        ones_v[pl.ds(q * L, L)] = one16

    def zb(i, carry):
        buf_v[pl.ds(i * L, L)] = zero16
        return carry

    lax.fori_loop(0, SLAB // L, zb, 0)
    pltpu.sync_copy(buf_v, acc_sh.at[pl.ds(sid * SLAB, SLAB)])
    plsc.subcore_barrier()

    def chunk(c, carry):
        row0 = pl.multiple_of(wid * (EW // 128) + c * IR, IR)
        pltpu.sync_copy(dst2d_hbm.at[pl.ds(row0, IR)], didx_v)
        descs = [
            pltpu.async_copy(ones_v, acc_sh.at[didx_v.at[j]], ssem, add=True)
            for j in range(IR)
        ]
        for d in descs:
            d.wait()
        return carry

    lax.fori_loop(0, NCHUNK, chunk, 0)
    plsc.subcore_barrier()
    pltpu.sync_copy(acc_sh.at[pl.ds(sid * SLAB, SLAB)], buf_v)
    pltpu.sync_copy(buf_v, out_hbm.at[pl.ds(cid * NP + sid * SLAB, SLAB)])


def _make_agg_kernel(half):
    @functools.partial(
        pl.kernel,
        out_type=jax.ShapeDtypeStruct((NC, NP, 32), jnp.float32),
        mesh=_mesh,
        compiler_params=pltpu.CompilerParams(use_tc_tiling_on_sc=False),
        scratch_types=[
            pltpu.VMEM((IR, 128), jnp.int32),
            pltpu.VMEM((IR, 128), jnp.int32),
            pltpu.VMEM((CK, 32), jnp.float32),
            pltpu.VMEM_SHARED((NP, 32), jnp.float32),
            pltpu.SemaphoreType.DMA,
            pltpu.SemaphoreType.DMA,
            pltpu.SemaphoreType.DMA,
        ],
    )
    def _agg(y2_hbm, src2d_hbm, dst2d_hbm, out_hbm,
             sidx_v, didx_v, rows_v, acc_sh, gsem, ssem, wsem):
        cid = lax.axis_index("c")
        sid = lax.axis_index("s")
        wid = cid * NS + sid

        zero16 = jnp.zeros((L,), jnp.float32)

        def zb(r, carry):
            rows_v[r, pl.ds(0, L)] = zero16
            rows_v[r, pl.ds(L, L)] = zero16
            return carry

        lax.fori_loop(0, CK, zb, 0)
        pieces = [(o, CK) for o in range(0, (SLAB // CK) * CK, CK)]
        if SLAB % CK:
            pieces.append(((SLAB // CK) * CK, SLAB % CK))
        for off, sz in pieces:
            pltpu.sync_copy(rows_v.at[pl.ds(0, sz)],
                            acc_sh.at[pl.ds(sid * SLAB + off, sz)])
        plsc.subcore_barrier()

        def chunk(c, carry):
            row0 = pl.multiple_of(wid * (EW // 128) + c * IR, IR)
            pltpu.async_copy(src2d_hbm.at[pl.ds(row0, IR)], sidx_v, wsem)
            pltpu.async_copy(dst2d_hbm.at[pl.ds(row0, IR)], didx_v, wsem)
            pltpu.make_async_copy(src2d_hbm.at[pl.ds(row0, IR)], sidx_v, wsem).wait()
            pltpu.make_async_copy(dst2d_hbm.at[pl.ds(row0, IR)], didx_v, wsem).wait()
            for j in range(IR):
                for q in range(128 // L):
                    s = sidx_v[j, pl.ds(q * L, L)]
                    sidx_v[j, pl.ds(q * L, L)] = s * 2 + half
            gd = [
                pltpu.async_copy(y2_hbm.at[sidx_v.at[j]],
                                 rows_v.at[pl.ds(j * 128, 128)], gsem)
                for j in range(IR)
            ]
            for d in gd:
                d.wait()
            sd = [
                pltpu.async_copy(rows_v.at[pl.ds(j * 128, 128)],
                                 acc_sh.at[didx_v.at[j]], ssem, add=True)
                for j in range(IR)
            ]
            for d in sd:
                d.wait()
            return carry

        lax.fori_loop(0, NCHUNK, chunk, 0)
        plsc.subcore_barrier()
        for off, sz in pieces:
            pltpu.sync_copy(acc_sh.at[pl.ds(sid * SLAB + off, sz)],
                            rows_v.at[pl.ds(0, sz)])
            pltpu.sync_copy(rows_v.at[pl.ds(0, sz)],
                            out_hbm.at[cid, pl.ds(sid * SLAB + off, sz)])

    return _agg


_agg_kernel_h0 = _make_agg_kernel(0)
_agg_kernel_h1 = _make_agg_kernel(1)


_PCHUNK = P // NW // 128


@functools.partial(
    pl.kernel,
    out_type=(jax.ShapeDtypeStruct((P, D), jnp.float32),
              jax.ShapeDtypeStruct((P, D), jnp.float32)),
    mesh=_mesh,
    compiler_params=pltpu.CompilerParams(use_tc_tiling_on_sc=False),
    scratch_types=[
        pltpu.VMEM((128,), jnp.int32),
        pltpu.VMEM((128,), jnp.int32),
        pltpu.VMEM((128, D), jnp.float32),
        pltpu.VMEM((128, D), jnp.float32),
        pltpu.SemaphoreType.DMA,
        pltpu.SemaphoreType.DMA,
    ],
)
def _pair_gather_kernel(h_hbm, pidx_hbm, u_hbm, v_hbm,
                        uidx_v, vidx_v, ubuf_v, vbuf_v, gsem, wsem):
    wid = _wid()

    def chunk(c, carry):
        row = wid * _PCHUNK + c
        off = pl.multiple_of(row * 128, 8)
        pltpu.async_copy(pidx_hbm.at[pl.ds(off, 128)], uidx_v, wsem)
        pltpu.async_copy(pidx_hbm.at[pl.ds(P + off, 128)], vidx_v, wsem)
        pltpu.make_async_copy(pidx_hbm.at[pl.ds(off, 128)], uidx_v, wsem).wait()
        pltpu.make_async_copy(pidx_hbm.at[pl.ds(P + off, 128)], vidx_v, wsem).wait()
        gu = pltpu.async_copy(h_hbm.at[uidx_v], ubuf_v, gsem)
        gv = pltpu.async_copy(h_hbm.at[vidx_v], vbuf_v, gsem)
        gu.wait()
        gv.wait()
        wu = pltpu.async_copy(ubuf_v, u_hbm.at[pl.ds(off, 128)], wsem)
        wv = pltpu.async_copy(vbuf_v, v_hbm.at[pl.ds(off, 128)], wsem)
        wu.wait()
        wv.wait()
        return carry

    lax.fori_loop(0, _PCHUNK, chunk, 0)


_RA = 2000


def _tc_a_body(emb, W1, b1, d0, d1, y1, s1, dis, inv):
    deg = d0[...] + d1[...] + 1.0
    di = lax.rsqrt(deg)
    iv = 1.0 / deg
    xw = jnp.dot(emb[...], W1[...], preferred_element_type=jnp.float32)
    y1[...] = xw * di
    s1[...] = xw * iv + b1[...]
    dis[...] = di
    inv[...] = iv


def _tc_b_body(a00, a01, a10, a11, s1, dis, inv, W2, b2, y2, s2):
    agg = jnp.concatenate([a00[...] + a01[...], a10[...] + a11[...]], axis=1)
    h1 = jnp.maximum(dis[...] * agg + s1[...], 0.0)
    xw = jnp.dot(h1, W2[...], preferred_element_type=jnp.float32)
    y2[...] = xw * dis[...]
    s2[...] = xw * inv[...] + b2[...]


def _tc_c_body(a00, a01, a10, a11, s2, dis, h):
    agg = jnp.concatenate([a00[...] + a01[...], a10[...] + a11[...]], axis=1)
    h[...] = dis[...] * agg + s2[...]


_RD = 4096


def _tc_d_body(u, v, mW1, mb1, mW2, mb2, out):
    uu = u[...]
    vv = v[...]
    A = mW1[0:D, :]
    B = mW1[D:2 * D, :]
    C = mW1[2 * D:3 * D, :]
    Dm = mW1[3 * D:4 * D, :]
    t = (jnp.dot(uu, A, preferred_element_type=jnp.float32)
         + jnp.dot(vv, B, preferred_element_type=jnp.float32)
         + jnp.dot(jnp.abs(uu - vv), C, preferred_element_type=jnp.float32)
         + jnp.dot(uu * vv, Dm, preferred_element_type=jnp.float32)
         + mb1[...])
    hid = jnp.maximum(t, 0.0)
    out[...] = jnp.dot(hid, mW2[...], preferred_element_type=jnp.float32) + mb2[...]


def _row_spec(r, cols):
    return pl.BlockSpec((r, cols), lambda i: (i, 0))


def _full_spec(shape):
    nd = len(shape)
    return pl.BlockSpec(shape, lambda i: (0,) * nd)


def kernel(edge_index, pairs, emb, W1, b1, W2, b2, mW1, mb1, mW2, mb2):
    f32 = jnp.float32
    src = edge_index[0]
    dst = edge_index[1]
    pad = EP - E
    src_p = jnp.concatenate([src, jnp.zeros((pad,), jnp.int32)])
    dst_p = jnp.concatenate([dst, jnp.full((pad,), N, jnp.int32)])
    src2d = src_p.reshape(EP // 128, 128)
    dst2d = dst_p.reshape(EP // 128, 128)
    pidx = pairs.T.reshape(2 * P)
    b1r = b1.reshape(1, D)
    b2r = b2.reshape(1, D)
    mb1r = mb1.reshape(1, D)
    mb2r = mb2.reshape(1, 1)

    degp = _deg_kernel(dst2d)
    d0 = degp[:N].reshape(N, 1)
    d1 = degp[NP:NP + N].reshape(N, 1)

    grid_a = (N // _RA,)
    y1, s1, dis, inv = pl.pallas_call(
        _tc_a_body,
        grid=grid_a,
        in_specs=[_row_spec(_RA, D), _full_spec((D, D)), _full_spec((1, D)),
                  _row_spec(_RA, 1), _row_spec(_RA, 1)],
        out_specs=[_row_spec(_RA, D), _row_spec(_RA, D),
                   _row_spec(_RA, 1), _row_spec(_RA, 1)],
        out_shape=[jax.ShapeDtypeStruct((N, D), f32),
                   jax.ShapeDtypeStruct((N, D), f32),
                   jax.ShapeDtypeStruct((N, 1), f32),
                   jax.ShapeDtypeStruct((N, 1), f32)],
    )(emb, W1, b1r, d0, d1)

    y1r = y1.reshape(2 * N, 32)
    g0 = _agg_kernel_h0(y1r, src2d, dst2d)
    g1 = _agg_kernel_h1(y1r, src2d, dst2d)
    a00, a01 = g0[0, :N], g0[1, :N]
    a10, a11 = g1[0, :N], g1[1, :N]

    y2, s2 = pl.pallas_call(
        _tc_b_body,
        grid=grid_a,
        in_specs=[_row_spec(_RA, 32)] * 4
        + [_row_spec(_RA, D), _row_spec(_RA, 1), _row_spec(_RA, 1),
           _full_spec((D, D)), _full_spec((1, D))],
        out_specs=[_row_spec(_RA, D), _row_spec(_RA, D)],
        out_shape=[jax.ShapeDtypeStruct((N, D), f32),
                   jax.ShapeDtypeStruct((N, D), f32)],
    )(a00, a01, a10, a11, s1, dis, inv, W2, b2r)

    y2r = y2.reshape(2 * N, 32)
    g0 = _agg_kernel_h0(y2r, src2d, dst2d)
    g1 = _agg_kernel_h1(y2r, src2d, dst2d)
    a00, a01 = g0[0, :N], g0[1, :N]
    a10, a11 = g1[0, :N], g1[1, :N]

    h = pl.pallas_call(
        _tc_c_body,
        grid=grid_a,
        in_specs=[_row_spec(_RA, 32)] * 4 + [_row_spec(_RA, D), _row_spec(_RA, 1)],
        out_specs=_row_spec(_RA, D),
        out_shape=jax.ShapeDtypeStruct((N, D), f32),
    )(a00, a01, a10, a11, s2, dis)

    u, v = _pair_gather_kernel(h, pidx)

    logits = pl.pallas_call(
        _tc_d_body,
        grid=(P // _RD,),
        in_specs=[_row_spec(_RD, D), _row_spec(_RD, D),
                  _full_spec((4 * D, D)), _full_spec((1, D)),
                  _full_spec((D, 1)), _full_spec((1, 1))],
        out_specs=_row_spec(_RD, 1),
        out_shape=jax.ShapeDtypeStruct((P, 1), f32),
    )(u, v, mW1, mb1r, mW2, mb2r)
    return logits.reshape(P)


# speedup vs baseline: 19.2168x; 1.6617x over previous
import functools

import jax
import jax.numpy as jnp
from jax import lax
from jax.experimental import pallas as pl
from jax.experimental.pallas import tpu as pltpu
from jax.experimental.pallas import tpu_sc as plsc

N = 50000
D = 64
E = 800000
P = 65536

NC = 2
NS = 16
NW = NC * NS
L = 16

CK = 512
EW = 25600
EP = NW * EW
NCHUNK = EW // CK
NP = 50176
SLAB = NP // NS
IR = CK // 128

_mesh = plsc.VectorSubcoreMesh(core_axis_name="c", subcore_axis_name="s")


def _wid():
    return lax.axis_index("c") * NS + lax.axis_index("s")


@functools.partial(
    pl.kernel,
    out_type=jax.ShapeDtypeStruct((NC * NP,), jnp.float32),
    mesh=_mesh,
    compiler_params=pltpu.CompilerParams(use_tc_tiling_on_sc=False),
    scratch_types=[
        pltpu.VMEM((IR, 128), jnp.int32),
        pltpu.VMEM((128,), jnp.float32),
        pltpu.VMEM((SLAB,), jnp.float32),
        pltpu.VMEM_SHARED((NP,), jnp.float32),
        pltpu.SemaphoreType.DMA,
        pltpu.SemaphoreType.DMA,
    ],
)
def _deg_kernel(dst2d_hbm, out_hbm, didx_v, ones_v, buf_v, acc_sh, ssem, wsem):
    cid = lax.axis_index("c")
    sid = lax.axis_index("s")
    wid = cid * NS + sid

    zero16 = jnp.zeros((L,), jnp.float32)
    one16 = jnp.ones((L,), jnp.float32)
    for q in range(128 // L):
        ones_v[pl.ds(q * L, L)] = one16

    def zb(i, carry):
        buf_v[pl.ds(i * L, L)] = zero16
        return carry

    lax.fori_loop(0, SLAB // L, zb, 0)
    pltpu.sync_copy(buf_v, acc_sh.at[pl.ds(sid * SLAB, SLAB)])
    plsc.subcore_barrier()

    def chunk(c, carry):
        row0 = pl.multiple_of(wid * (EW // 128) + c * IR, IR)
        pltpu.sync_copy(dst2d_hbm.at[pl.ds(row0, IR)], didx_v)
        descs = [
            pltpu.async_copy(ones_v, acc_sh.at[didx_v.at[j]], ssem, add=True)
            for j in range(IR)
        ]
        for d in descs:
            d.wait()
        return carry

    lax.fori_loop(0, NCHUNK, chunk, 0)
    plsc.subcore_barrier()
    pltpu.sync_copy(acc_sh.at[pl.ds(sid * SLAB, SLAB)], buf_v)
    pltpu.sync_copy(buf_v, out_hbm.at[pl.ds(cid * NP + sid * SLAB, SLAB)])


def _make_agg_kernel(half):
    @functools.partial(
        pl.kernel,
        out_type=jax.ShapeDtypeStruct((NC, NP, 32), jnp.float32),
        mesh=_mesh,
        compiler_params=pltpu.CompilerParams(use_tc_tiling_on_sc=False),
        scratch_types=[
            pltpu.VMEM((IR, 128), jnp.int32),
            pltpu.VMEM((IR, 128), jnp.int32),
            pltpu.VMEM((CK, 32), jnp.float32),
            pltpu.VMEM_SHARED((NP, 32), jnp.float32),
            pltpu.SemaphoreType.DMA,
            pltpu.SemaphoreType.DMA,
            pltpu.SemaphoreType.DMA,
        ],
    )
    def _agg(y2_hbm, src2d_hbm, dst2d_hbm, out_hbm,
             sidx_v, didx_v, rows_v, acc_sh, gsem, ssem, wsem):
        cid = lax.axis_index("c")
        sid = lax.axis_index("s")
        wid = cid * NS + sid

        zero16 = jnp.zeros((L,), jnp.float32)

        def zb(r, carry):
            rows_v[r, pl.ds(0, L)] = zero16
            rows_v[r, pl.ds(L, L)] = zero16
            return carry

        lax.fori_loop(0, CK, zb, 0)
        pieces = [(o, CK) for o in range(0, (SLAB // CK) * CK, CK)]
        if SLAB % CK:
            pieces.append(((SLAB // CK) * CK, SLAB % CK))
        for off, sz in pieces:
            pltpu.sync_copy(rows_v.at[pl.ds(0, sz)],
                            acc_sh.at[pl.ds(sid * SLAB + off, sz)])
        plsc.subcore_barrier()

        def chunk(c, carry):
            row0 = pl.multiple_of(wid * (EW // 128) + c * IR, IR)
            pltpu.async_copy(src2d_hbm.at[pl.ds(row0, IR)], sidx_v, wsem)
            pltpu.async_copy(dst2d_hbm.at[pl.ds(row0, IR)], didx_v, wsem)
            pltpu.make_async_copy(src2d_hbm.at[pl.ds(row0, IR)], sidx_v, wsem).wait()
            pltpu.make_async_copy(dst2d_hbm.at[pl.ds(row0, IR)], didx_v, wsem).wait()
            for j in range(IR):
                for q in range(128 // L):
                    s = sidx_v[j, pl.ds(q * L, L)]
                    sidx_v[j, pl.ds(q * L, L)] = s * 2 + half
            gd = [
                pltpu.async_copy(y2_hbm.at[sidx_v.at[j]],
                                 rows_v.at[pl.ds(j * 128, 128)], gsem)
                for j in range(IR)
            ]
            for d in gd:
                d.wait()
            sd = [
                pltpu.async_copy(rows_v.at[pl.ds(j * 128, 128)],
                                 acc_sh.at[didx_v.at[j]], ssem, add=True)
                for j in range(IR)
            ]
            for d in sd:
                d.wait()
            return carry

        lax.fori_loop(0, NCHUNK, chunk, 0)
        plsc.subcore_barrier()
        for off, sz in pieces:
            pltpu.sync_copy(acc_sh.at[pl.ds(sid * SLAB + off, sz)],
                            rows_v.at[pl.ds(0, sz)])
            pltpu.sync_copy(rows_v.at[pl.ds(0, sz)],
                            out_hbm.at[cid, pl.ds(sid * SLAB + off, sz)])

    return _agg


_agg_kernel_h0 = _make_agg_kernel(0)
_agg_kernel_h1 = _make_agg_kernel(1)


_PCHUNK = P // NW // 128


@functools.partial(
    pl.kernel,
    out_type=(jax.ShapeDtypeStruct((P, D), jnp.float32),
              jax.ShapeDtypeStruct((P, D), jnp.float32)),
    mesh=_mesh,
    compiler_params=pltpu.CompilerParams(use_tc_tiling_on_sc=False),
    scratch_types=[
        pltpu.VMEM((128,), jnp.int32),
        pltpu.VMEM((128,), jnp.int32),
        pltpu.VMEM((128, D), jnp.float32),
        pltpu.VMEM((128, D), jnp.float32),
        pltpu.SemaphoreType.DMA,
        pltpu.SemaphoreType.DMA,
    ],
)
def _pair_gather_kernel(h_hbm, pidx_hbm, u_hbm, v_hbm,
                        uidx_v, vidx_v, ubuf_v, vbuf_v, gsem, wsem):
    wid = _wid()

    def chunk(c, carry):
        row = wid * _PCHUNK + c
        off = pl.multiple_of(row * 128, 8)
        pltpu.async_copy(pidx_hbm.at[pl.ds(off, 128)], uidx_v, wsem)
        pltpu.async_copy(pidx_hbm.at[pl.ds(P + off, 128)], vidx_v, wsem)
        pltpu.make_async_copy(pidx_hbm.at[pl.ds(off, 128)], uidx_v, wsem).wait()
        pltpu.make_async_copy(pidx_hbm.at[pl.ds(P + off, 128)], vidx_v, wsem).wait()
        gu = pltpu.async_copy(h_hbm.at[uidx_v], ubuf_v, gsem)
        gv = pltpu.async_copy(h_hbm.at[vidx_v], vbuf_v, gsem)
        gu.wait()
        gv.wait()
        wu = pltpu.async_copy(ubuf_v, u_hbm.at[pl.ds(off, 128)], wsem)
        wv = pltpu.async_copy(vbuf_v, v_hbm.at[pl.ds(off, 128)], wsem)
        wu.wait()
        wv.wait()
        return carry

    lax.fori_loop(0, _PCHUNK, chunk, 0)


_RA = 2000


def _tc_a_body(emb, W1, b1, d0, d1, y1, s1, dis, inv):
    deg = d0[...] + d1[...] + 1.0
    di = lax.rsqrt(deg)
    iv = 1.0 / deg
    xw = jnp.dot(emb[...], W1[...], preferred_element_type=jnp.float32)
    y1[...] = xw * di
    s1[...] = xw * iv + b1[...]
    dis[...] = di
    inv[...] = iv


def _tc_b_body(a00, a01, a10, a11, s1, dis, inv, W2, b2, y2, s2):
    agg = jnp.concatenate([a00[...] + a01[...], a10[...] + a11[...]], axis=1)
    h1 = jnp.maximum(dis[...] * agg + s1[...], 0.0)
    xw = jnp.dot(h1, W2[...], preferred_element_type=jnp.float32)
    y2[...] = xw * dis[...]
    s2[...] = xw * inv[...] + b2[...]


def _tc_c_body(a00, a01, a10, a11, s2, dis, h):
    agg = jnp.concatenate([a00[...] + a01[...], a10[...] + a11[...]], axis=1)
    h[...] = dis[...] * agg + s2[...]


_RD = 4096


def _tc_d_body(u, v, mW1, mb1, mW2, mb2, out):
    uu = u[...]
    vv = v[...]
    A = mW1[0:D, :]
    B = mW1[D:2 * D, :]
    C = mW1[2 * D:3 * D, :]
    Dm = mW1[3 * D:4 * D, :]
    t = (jnp.dot(uu, A, preferred_element_type=jnp.float32)
         + jnp.dot(vv, B, preferred_element_type=jnp.float32)
         + jnp.dot(jnp.abs(uu - vv), C, preferred_element_type=jnp.float32)
         + jnp.dot(uu * vv, Dm, preferred_element_type=jnp.float32)
         + mb1[...])
    hid = jnp.maximum(t, 0.0)
    out[...] = jnp.dot(hid, mW2[...], preferred_element_type=jnp.float32) + mb2[...]


def _row_spec(r, cols):
    return pl.BlockSpec((r, cols), lambda i: (i, 0))


def _full_spec(shape):
    nd = len(shape)
    return pl.BlockSpec(shape, lambda i: (0,) * nd)


def kernel(edge_index, pairs, emb, W1, b1, W2, b2, mW1, mb1, mW2, mb2):
    f32 = jnp.float32
    src = edge_index[0]
    dst = edge_index[1]
    pad = EP - E
    pad_iota = jnp.arange(pad, dtype=jnp.int32)
    src_p = jnp.concatenate([src, pad_iota % N])
    dst_p = jnp.concatenate([dst, N + pad_iota % (NP - N)])
    src2d = src_p.reshape(EP // 128, 128)
    dst2d = dst_p.reshape(EP // 128, 128)
    pidx = pairs.T.reshape(2 * P)
    b1r = b1.reshape(1, D)
    b2r = b2.reshape(1, D)
    mb1r = mb1.reshape(1, D)
    mb2r = mb2.reshape(1, 1)

    degp = _deg_kernel(dst2d)
    d0 = degp[:N].reshape(N, 1)
    d1 = degp[NP:NP + N].reshape(N, 1)

    grid_a = (N // _RA,)
    y1, s1, dis, inv = pl.pallas_call(
        _tc_a_body,
        grid=grid_a,
        in_specs=[_row_spec(_RA, D), _full_spec((D, D)), _full_spec((1, D)),
                  _row_spec(_RA, 1), _row_spec(_RA, 1)],
        out_specs=[_row_spec(_RA, D), _row_spec(_RA, D),
                   _row_spec(_RA, 1), _row_spec(_RA, 1)],
        out_shape=[jax.ShapeDtypeStruct((N, D), f32),
                   jax.ShapeDtypeStruct((N, D), f32),
                   jax.ShapeDtypeStruct((N, 1), f32),
                   jax.ShapeDtypeStruct((N, 1), f32)],
    )(emb, W1, b1r, d0, d1)

    y1r = y1.reshape(2 * N, 32)
    g0 = _agg_kernel_h0(y1r, src2d, dst2d)
    g1 = _agg_kernel_h1(y1r, src2d, dst2d)
    a00, a01 = g0[0, :N], g0[1, :N]
    a10, a11 = g1[0, :N], g1[1, :N]

    y2, s2 = pl.pallas_call(
        _tc_b_body,
        grid=grid_a,
        in_specs=[_row_spec(_RA, 32)] * 4
        + [_row_spec(_RA, D), _row_spec(_RA, 1), _row_spec(_RA, 1),
           _full_spec((D, D)), _full_spec((1, D))],
        out_specs=[_row_spec(_RA, D), _row_spec(_RA, D)],
        out_shape=[jax.ShapeDtypeStruct((N, D), f32),
                   jax.ShapeDtypeStruct((N, D), f32)],
    )(a00, a01, a10, a11, s1, dis, inv, W2, b2r)

    y2r = y2.reshape(2 * N, 32)
    g0 = _agg_kernel_h0(y2r, src2d, dst2d)
    g1 = _agg_kernel_h1(y2r, src2d, dst2d)
    a00, a01 = g0[0, :N], g0[1, :N]
    a10, a11 = g1[0, :N], g1[1, :N]

    h = pl.pallas_call(
        _tc_c_body,
        grid=grid_a,
        in_specs=[_row_spec(_RA, 32)] * 4 + [_row_spec(_RA, D), _row_spec(_RA, 1)],
        out_specs=_row_spec(_RA, D),
        out_shape=jax.ShapeDtypeStruct((N, D), f32),
    )(a00, a01, a10, a11, s2, dis)

    u, v = _pair_gather_kernel(h, pidx)

    logits = pl.pallas_call(
        _tc_d_body,
        grid=(P // _RD,),
        in_specs=[_row_spec(_RD, D), _row_spec(_RD, D),
                  _full_spec((4 * D, D)), _full_spec((1, D)),
                  _full_spec((D, 1)), _full_spec((1, 1))],
        out_specs=_row_spec(_RD, 1),
        out_shape=jax.ShapeDtypeStruct((P, 1), f32),
    )(u, v, mW1, mb1r, mW2, mb2r)
    return logits.reshape(P)


# speedup vs baseline: 22.3607x; 1.1636x over previous
import functools

import jax
import jax.numpy as jnp
from jax import lax
from jax.experimental import pallas as pl
from jax.experimental.pallas import tpu as pltpu
from jax.experimental.pallas import tpu_sc as plsc

N = 50000
D = 64
E = 800000
P = 65536

NC = 2
NS = 16
NW = NC * NS
L = 16

CK = 512
EW = 25600
EP = NW * EW
NCHUNK = EW // CK
NP = 50176
SLAB = NP // NS
IR = CK // 128

_mesh = plsc.VectorSubcoreMesh(core_axis_name="c", subcore_axis_name="s")


def _wid():
    return lax.axis_index("c") * NS + lax.axis_index("s")


@functools.partial(
    pl.kernel,
    out_type=jax.ShapeDtypeStruct((NC * NP,), jnp.float32),
    mesh=_mesh,
    compiler_params=pltpu.CompilerParams(use_tc_tiling_on_sc=False),
    scratch_types=[
        pltpu.VMEM((IR, 128), jnp.int32),
        pltpu.VMEM((128,), jnp.float32),
        pltpu.VMEM((SLAB,), jnp.float32),
        pltpu.VMEM_SHARED((NP,), jnp.float32),
        pltpu.SemaphoreType.DMA,
        pltpu.SemaphoreType.DMA,
    ],
)
def _deg_kernel(dst2d_hbm, out_hbm, didx_v, ones_v, buf_v, acc_sh, ssem, wsem):
    cid = lax.axis_index("c")
    sid = lax.axis_index("s")
    wid = cid * NS + sid

    zero16 = jnp.zeros((L,), jnp.float32)
    one16 = jnp.ones((L,), jnp.float32)
    for q in range(128 // L):
        ones_v[pl.ds(q * L, L)] = one16

    def zb(i, carry):
        buf_v[pl.ds(i * L, L)] = zero16
        return carry

    lax.fori_loop(0, SLAB // L, zb, 0)
    pltpu.sync_copy(buf_v, acc_sh.at[pl.ds(sid * SLAB, SLAB)])
    plsc.subcore_barrier()

    def chunk(c, carry):
        row0 = pl.multiple_of(wid * (EW // 128) + c * IR, IR)
        pltpu.sync_copy(dst2d_hbm.at[pl.ds(row0, IR)], didx_v)
        descs = [
            pltpu.async_copy(ones_v, acc_sh.at[didx_v.at[j]], ssem, add=True)
            for j in range(IR)
        ]
        for d in descs:
            d.wait()
        return carry

    lax.fori_loop(0, NCHUNK, chunk, 0)
    plsc.subcore_barrier()
    pltpu.sync_copy(acc_sh.at[pl.ds(sid * SLAB, SLAB)], buf_v)
    pltpu.sync_copy(buf_v, out_hbm.at[pl.ds(cid * NP + sid * SLAB, SLAB)])


EW2 = EP // NS
NCHUNK2 = EW2 // CK


@functools.partial(
    pl.kernel,
    out_type=jax.ShapeDtypeStruct((NC, NP, 32), jnp.float32),
    mesh=_mesh,
    compiler_params=pltpu.CompilerParams(use_tc_tiling_on_sc=False),
    scratch_types=[
        pltpu.VMEM((IR, 128), jnp.int32),
        pltpu.VMEM((IR, 128), jnp.int32),
        pltpu.VMEM((CK, 32), jnp.float32),
        pltpu.VMEM_SHARED((NP, 32), jnp.float32),
        pltpu.SemaphoreType.DMA,
        pltpu.SemaphoreType.DMA,
        pltpu.SemaphoreType.DMA,
    ],
)
def _agg_kernel(y2_hbm, srch_hbm, dst2d_hbm, out_hbm,
                sidx_v, didx_v, rows_v, acc_sh, gsem, ssem, wsem):
    cid = lax.axis_index("c")
    sid = lax.axis_index("s")

    zero16 = jnp.zeros((L,), jnp.float32)

    def zb(r, carry):
        rows_v[r, pl.ds(0, L)] = zero16
        rows_v[r, pl.ds(L, L)] = zero16
        return carry

    lax.fori_loop(0, CK, zb, 0)
    pieces = [(o, CK) for o in range(0, (SLAB // CK) * CK, CK)]
    if SLAB % CK:
        pieces.append(((SLAB // CK) * CK, SLAB % CK))
    for off, sz in pieces:
        pltpu.sync_copy(rows_v.at[pl.ds(0, sz)],
                        acc_sh.at[pl.ds(sid * SLAB + off, sz)])
    plsc.subcore_barrier()

    def chunk(c, carry):
        row0 = pl.multiple_of(sid * (EW2 // 128) + c * IR, IR)
        pltpu.async_copy(srch_hbm.at[cid, pl.ds(row0, IR)], sidx_v, wsem)
        pltpu.async_copy(dst2d_hbm.at[pl.ds(row0, IR)], didx_v, wsem)
        pltpu.make_async_copy(srch_hbm.at[cid, pl.ds(row0, IR)], sidx_v, wsem).wait()
        pltpu.make_async_copy(dst2d_hbm.at[pl.ds(row0, IR)], didx_v, wsem).wait()
        gd = [
            pltpu.async_copy(y2_hbm.at[sidx_v.at[j]],
                             rows_v.at[pl.ds(j * 128, 128)], gsem)
            for j in range(IR)
        ]
        sd = []
        for j in range(IR):
            gd[j].wait()
            sd.append(pltpu.async_copy(rows_v.at[pl.ds(j * 128, 128)],
                                       acc_sh.at[didx_v.at[j]], ssem, add=True))
        for d in sd:
            d.wait()
        return carry

    lax.fori_loop(0, NCHUNK2, chunk, 0)
    plsc.subcore_barrier()
    for off, sz in pieces:
        pltpu.sync_copy(acc_sh.at[pl.ds(sid * SLAB + off, sz)],
                        rows_v.at[pl.ds(0, sz)])
        pltpu.sync_copy(rows_v.at[pl.ds(0, sz)],
                        out_hbm.at[cid, pl.ds(sid * SLAB + off, sz)])


_PCHUNK = P // NW // 128


@functools.partial(
    pl.kernel,
    out_type=(jax.ShapeDtypeStruct((P, D), jnp.float32),
              jax.ShapeDtypeStruct((P, D), jnp.float32)),
    mesh=_mesh,
    compiler_params=pltpu.CompilerParams(use_tc_tiling_on_sc=False),
    scratch_types=[
        pltpu.VMEM((128,), jnp.int32),
        pltpu.VMEM((128,), jnp.int32),
        pltpu.VMEM((128, D), jnp.float32),
        pltpu.VMEM((128, D), jnp.float32),
        pltpu.SemaphoreType.DMA,
        pltpu.SemaphoreType.DMA,
    ],
)
def _pair_gather_kernel(h_hbm, pidx_hbm, u_hbm, v_hbm,
                        uidx_v, vidx_v, ubuf_v, vbuf_v, gsem, wsem):
    wid = _wid()

    def chunk(c, carry):
        row = wid * _PCHUNK + c
        off = pl.multiple_of(row * 128, 8)
        pltpu.async_copy(pidx_hbm.at[pl.ds(off, 128)], uidx_v, wsem)
        pltpu.async_copy(pidx_hbm.at[pl.ds(P + off, 128)], vidx_v, wsem)
        pltpu.make_async_copy(pidx_hbm.at[pl.ds(off, 128)], uidx_v, wsem).wait()
        pltpu.make_async_copy(pidx_hbm.at[pl.ds(P + off, 128)], vidx_v, wsem).wait()
        gu = pltpu.async_copy(h_hbm.at[uidx_v], ubuf_v, gsem)
        gv = pltpu.async_copy(h_hbm.at[vidx_v], vbuf_v, gsem)
        gu.wait()
        gv.wait()
        wu = pltpu.async_copy(ubuf_v, u_hbm.at[pl.ds(off, 128)], wsem)
        wv = pltpu.async_copy(vbuf_v, v_hbm.at[pl.ds(off, 128)], wsem)
        wu.wait()
        wv.wait()
        return carry

    lax.fori_loop(0, _PCHUNK, chunk, 0)


_RA = 2000


def _tc_a_body(emb, W1, b1, d0, d1, y1, s1, dis, inv):
    deg = d0[...] + d1[...] + 1.0
    di = lax.rsqrt(deg)
    iv = 1.0 / deg
    xw = jnp.dot(emb[...], W1[...], preferred_element_type=jnp.float32)
    y1[...] = xw * di
    s1[...] = xw * iv + b1[...]
    dis[...] = di
    inv[...] = iv


def _tc_b_body(g0, g1, s1, dis, inv, W2, b2, y2, s2):
    agg = jnp.concatenate([g0[...], g1[...]], axis=1)
    h1 = jnp.maximum(dis[...] * agg + s1[...], 0.0)
    xw = jnp.dot(h1, W2[...], preferred_element_type=jnp.float32)
    y2[...] = xw * dis[...]
    s2[...] = xw * inv[...] + b2[...]


def _tc_c_body(g0, g1, s2, dis, h):
    agg = jnp.concatenate([g0[...], g1[...]], axis=1)
    h[...] = dis[...] * agg + s2[...]


_RD = 4096


def _tc_d_body(u, v, mW1, mb1, mW2, mb2, out):
    uu = u[...]
    vv = v[...]
    A = mW1[0:D, :]
    B = mW1[D:2 * D, :]
    C = mW1[2 * D:3 * D, :]
    Dm = mW1[3 * D:4 * D, :]
    t = (jnp.dot(uu, A, preferred_element_type=jnp.float32)
         + jnp.dot(vv, B, preferred_element_type=jnp.float32)
         + jnp.dot(jnp.abs(uu - vv), C, preferred_element_type=jnp.float32)
         + jnp.dot(uu * vv, Dm, preferred_element_type=jnp.float32)
         + mb1[...])
    hid = jnp.maximum(t, 0.0)
    out[...] = jnp.dot(hid, mW2[...], preferred_element_type=jnp.float32) + mb2[...]


def _row_spec(r, cols):
    return pl.BlockSpec((r, cols), lambda i: (i, 0))


def _full_spec(shape):
    nd = len(shape)
    return pl.BlockSpec(shape, lambda i: (0,) * nd)


def kernel(edge_index, pairs, emb, W1, b1, W2, b2, mW1, mb1, mW2, mb2):
    f32 = jnp.float32
    src = edge_index[0]
    dst = edge_index[1]
    pad = EP - E
    pad_iota = jnp.arange(pad, dtype=jnp.int32)
    src_p = jnp.concatenate([src, pad_iota % N])
    dst_p = jnp.concatenate([dst, N + pad_iota % (NP - N)])
    srch = jnp.stack([src_p * 2, src_p * 2 + 1]).reshape(2, EP // 128, 128)
    dst2d = dst_p.reshape(EP // 128, 128)
    pidx = pairs.T.reshape(2 * P)
    b1r = b1.reshape(1, D)
    b2r = b2.reshape(1, D)
    mb1r = mb1.reshape(1, D)
    mb2r = mb2.reshape(1, 1)

    degp = _deg_kernel(dst2d)
    d0 = degp[:N].reshape(N, 1)
    d1 = degp[NP:NP + N].reshape(N, 1)

    grid_a = (N // _RA,)
    y1, s1, dis, inv = pl.pallas_call(
        _tc_a_body,
        grid=grid_a,
        in_specs=[_row_spec(_RA, D), _full_spec((D, D)), _full_spec((1, D)),
                  _row_spec(_RA, 1), _row_spec(_RA, 1)],
        out_specs=[_row_spec(_RA, D), _row_spec(_RA, D),
                   _row_spec(_RA, 1), _row_spec(_RA, 1)],
        out_shape=[jax.ShapeDtypeStruct((N, D), f32),
                   jax.ShapeDtypeStruct((N, D), f32),
                   jax.ShapeDtypeStruct((N, 1), f32),
                   jax.ShapeDtypeStruct((N, 1), f32)],
    )(emb, W1, b1r, d0, d1)

    y1r = y1.reshape(2 * N, 32)
    g = _agg_kernel(y1r, srch, dst2d)
    g0, g1 = g[0, :N], g[1, :N]

    y2, s2 = pl.pallas_call(
        _tc_b_body,
        grid=grid_a,
        in_specs=[_row_spec(_RA, 32)] * 2
        + [_row_spec(_RA, D), _row_spec(_RA, 1), _row_spec(_RA, 1),
           _full_spec((D, D)), _full_spec((1, D))],
        out_specs=[_row_spec(_RA, D), _row_spec(_RA, D)],
        out_shape=[jax.ShapeDtypeStruct((N, D), f32),
                   jax.ShapeDtypeStruct((N, D), f32)],
    )(g0, g1, s1, dis, inv, W2, b2r)

    y2r = y2.reshape(2 * N, 32)
    g = _agg_kernel(y2r, srch, dst2d)
    g0, g1 = g[0, :N], g[1, :N]

    h = pl.pallas_call(
        _tc_c_body,
        grid=grid_a,
        in_specs=[_row_spec(_RA, 32)] * 2 + [_row_spec(_RA, D), _row_spec(_RA, 1)],
        out_specs=_row_spec(_RA, D),
        out_shape=jax.ShapeDtypeStruct((N, D), f32),
    )(g0, g1, s2, dis)

    u, v = _pair_gather_kernel(h, pidx)

    logits = pl.pallas_call(
        _tc_d_body,
        grid=(P // _RD,),
        in_specs=[_row_spec(_RD, D), _row_spec(_RD, D),
                  _full_spec((4 * D, D)), _full_spec((1, D)),
                  _full_spec((D, 1)), _full_spec((1, 1))],
        out_specs=_row_spec(_RD, 1),
        out_shape=jax.ShapeDtypeStruct((P, 1), f32),
    )(u, v, mW1, mb1r, mW2, mb2r)
    return logits.reshape(P)


# speedup vs baseline: 24.6996x; 1.1046x over previous
import functools

import jax
import jax.numpy as jnp
from jax import lax
from jax.experimental import pallas as pl
from jax.experimental.pallas import tpu as pltpu
from jax.experimental.pallas import tpu_sc as plsc

N = 50000
D = 64
E = 800000
P = 65536

NC = 2
NS = 16
NW = NC * NS
L = 16

CK = 512
EW = 25600
EP = NW * EW
NCHUNK = EW // CK
NP = 50176
SLAB = NP // NS
IR = CK // 128

_mesh = plsc.VectorSubcoreMesh(core_axis_name="c", subcore_axis_name="s")


def _wid():
    return lax.axis_index("c") * NS + lax.axis_index("s")


@functools.partial(
    pl.kernel,
    out_type=jax.ShapeDtypeStruct((NC * NP,), jnp.float32),
    mesh=_mesh,
    compiler_params=pltpu.CompilerParams(use_tc_tiling_on_sc=False),
    scratch_types=[
        pltpu.VMEM((IR, 128), jnp.int32),
        pltpu.VMEM((128,), jnp.float32),
        pltpu.VMEM((SLAB,), jnp.float32),
        pltpu.VMEM_SHARED((NP,), jnp.float32),
        pltpu.SemaphoreType.DMA,
        pltpu.SemaphoreType.DMA,
    ],
)
def _deg_kernel(dst2d_hbm, out_hbm, didx_v, ones_v, buf_v, acc_sh, ssem, wsem):
    cid = lax.axis_index("c")
    sid = lax.axis_index("s")
    wid = cid * NS + sid

    zero16 = jnp.zeros((L,), jnp.float32)
    one16 = jnp.ones((L,), jnp.float32)
    for q in range(128 // L):
        ones_v[pl.ds(q * L, L)] = one16

    def zb(i, carry):
        buf_v[pl.ds(i * L, L)] = zero16
        return carry

    lax.fori_loop(0, SLAB // L, zb, 0)
    pltpu.sync_copy(buf_v, acc_sh.at[pl.ds(sid * SLAB, SLAB)])
    plsc.subcore_barrier()

    def chunk(c, carry):
        row0 = pl.multiple_of(wid * (EW // 128) + c * IR, IR)
        pltpu.sync_copy(dst2d_hbm.at[pl.ds(row0, IR)], didx_v)
        descs = [
            pltpu.async_copy(ones_v, acc_sh.at[didx_v.at[j]], ssem, add=True)
            for j in range(IR)
        ]
        for d in descs:
            d.wait()
        return carry

    lax.fori_loop(0, NCHUNK, chunk, 0)
    plsc.subcore_barrier()
    pltpu.sync_copy(acc_sh.at[pl.ds(sid * SLAB, SLAB)], buf_v)
    pltpu.sync_copy(buf_v, out_hbm.at[pl.ds(cid * NP + sid * SLAB, SLAB)])


EW2 = EP // NS
NCHUNK2 = EW2 // CK


@functools.partial(
    pl.kernel,
    out_type=jax.ShapeDtypeStruct((NC, NP, 32), jnp.float32),
    mesh=_mesh,
    compiler_params=pltpu.CompilerParams(use_tc_tiling_on_sc=False),
    scratch_types=[
        pltpu.VMEM((2, IR, 128), jnp.int32),
        pltpu.VMEM((2, IR, 128), jnp.int32),
        pltpu.VMEM((CK, 32), jnp.float32),
        pltpu.VMEM_SHARED((NP, 32), jnp.float32),
        pltpu.SemaphoreType.DMA,
        pltpu.SemaphoreType.DMA,
        pltpu.SemaphoreType.DMA,
    ],
)
def _agg_kernel(y2_hbm, srch_hbm, dst2d_hbm, out_hbm,
                sidx_v, didx_v, rows_v, acc_sh, gsem, ssem, wsem):
    cid = lax.axis_index("c")
    sid = lax.axis_index("s")

    zero16 = jnp.zeros((L,), jnp.float32)

    def zb(r, carry):
        rows_v[r, pl.ds(0, L)] = zero16
        rows_v[r, pl.ds(L, L)] = zero16
        return carry

    lax.fori_loop(0, CK, zb, 0)
    pieces = [(o, CK) for o in range(0, (SLAB // CK) * CK, CK)]
    if SLAB % CK:
        pieces.append(((SLAB // CK) * CK, SLAB % CK))
    for off, sz in pieces:
        pltpu.sync_copy(rows_v.at[pl.ds(0, sz)],
                        acc_sh.at[pl.ds(sid * SLAB + off, sz)])
    plsc.subcore_barrier()

    def _idx_rows(c):
        return pl.multiple_of(sid * (EW2 // 128) + c * IR, IR)

    def _issue_idx(c, par):
        r = _idx_rows(c)
        pltpu.async_copy(srch_hbm.at[cid, pl.ds(r, IR)], sidx_v.at[par], wsem)
        pltpu.async_copy(dst2d_hbm.at[pl.ds(r, IR)], didx_v.at[par], wsem)

    def _wait_idx(c, par):
        r = _idx_rows(c)
        pltpu.make_async_copy(srch_hbm.at[cid, pl.ds(r, IR)],
                              sidx_v.at[par], wsem).wait()
        pltpu.make_async_copy(dst2d_hbm.at[pl.ds(r, IR)],
                              didx_v.at[par], wsem).wait()

    _issue_idx(0, 0)

    def chunk2(i, carry):
        for sub in range(2):
            c = i * 2 + sub
            par = sub
            _wait_idx(c, par)
            _issue_idx(c + 1, 1 - par)
            gd = [
                pltpu.async_copy(y2_hbm.at[sidx_v.at[par, j]],
                                 rows_v.at[pl.ds(j * 128, 128)], gsem)
                for j in range(IR)
            ]
            sd = []
            for j in range(IR):
                gd[j].wait()
                sd.append(pltpu.async_copy(
                    rows_v.at[pl.ds(j * 128, 128)],
                    acc_sh.at[didx_v.at[par, j]], ssem, add=True))
            for d in sd:
                d.wait()
        return carry

    lax.fori_loop(0, NCHUNK2 // 2, chunk2, 0)
    _wait_idx(NCHUNK2, 0)
    plsc.subcore_barrier()
    for off, sz in pieces:
        pltpu.sync_copy(acc_sh.at[pl.ds(sid * SLAB + off, sz)],
                        rows_v.at[pl.ds(0, sz)])
        pltpu.sync_copy(rows_v.at[pl.ds(0, sz)],
                        out_hbm.at[cid, pl.ds(sid * SLAB + off, sz)])


_PCHUNK = P // NW // 128


@functools.partial(
    pl.kernel,
    out_type=(jax.ShapeDtypeStruct((P, D), jnp.float32),
              jax.ShapeDtypeStruct((P, D), jnp.float32)),
    mesh=_mesh,
    compiler_params=pltpu.CompilerParams(use_tc_tiling_on_sc=False),
    scratch_types=[
        pltpu.VMEM((128,), jnp.int32),
        pltpu.VMEM((128,), jnp.int32),
        pltpu.VMEM((128, D), jnp.float32),
        pltpu.VMEM((128, D), jnp.float32),
        pltpu.SemaphoreType.DMA,
        pltpu.SemaphoreType.DMA,
    ],
)
def _pair_gather_kernel(h_hbm, pidx_hbm, u_hbm, v_hbm,
                        uidx_v, vidx_v, ubuf_v, vbuf_v, gsem, wsem):
    wid = _wid()

    def chunk(c, carry):
        row = wid * _PCHUNK + c
        off = pl.multiple_of(row * 128, 8)
        pltpu.async_copy(pidx_hbm.at[pl.ds(off, 128)], uidx_v, wsem)
        pltpu.async_copy(pidx_hbm.at[pl.ds(P + off, 128)], vidx_v, wsem)
        pltpu.make_async_copy(pidx_hbm.at[pl.ds(off, 128)], uidx_v, wsem).wait()
        pltpu.make_async_copy(pidx_hbm.at[pl.ds(P + off, 128)], vidx_v, wsem).wait()
        gu = pltpu.async_copy(h_hbm.at[uidx_v], ubuf_v, gsem)
        gv = pltpu.async_copy(h_hbm.at[vidx_v], vbuf_v, gsem)
        gu.wait()
        gv.wait()
        wu = pltpu.async_copy(ubuf_v, u_hbm.at[pl.ds(off, 128)], wsem)
        wv = pltpu.async_copy(vbuf_v, v_hbm.at[pl.ds(off, 128)], wsem)
        wu.wait()
        wv.wait()
        return carry

    lax.fori_loop(0, _PCHUNK, chunk, 0)


_RA = 2000


def _tc_a_body(emb, W1, b1, d0, d1, y1, s1, dis, inv):
    deg = d0[...] + d1[...] + 1.0
    di = lax.rsqrt(deg)
    iv = 1.0 / deg
    xw = jnp.dot(emb[...], W1[...], preferred_element_type=jnp.float32)
    y1[...] = xw * di
    s1[...] = xw * iv + b1[...]
    dis[...] = di
    inv[...] = iv


def _tc_b_body(g0, g1, s1, dis, inv, W2, b2, y2, s2):
    agg = jnp.concatenate([g0[...], g1[...]], axis=1)
    h1 = jnp.maximum(dis[...] * agg + s1[...], 0.0)
    xw = jnp.dot(h1, W2[...], preferred_element_type=jnp.float32)
    y2[...] = xw * dis[...]
    s2[...] = xw * inv[...] + b2[...]


def _tc_c_body(g0, g1, s2, dis, h):
    agg = jnp.concatenate([g0[...], g1[...]], axis=1)
    h[...] = dis[...] * agg + s2[...]


_RD = 4096


def _tc_d_body(u, v, mW1, mb1, mW2, mb2, out):
    uu = u[...]
    vv = v[...]
    A = mW1[0:D, :]
    B = mW1[D:2 * D, :]
    C = mW1[2 * D:3 * D, :]
    Dm = mW1[3 * D:4 * D, :]
    t = (jnp.dot(uu, A, preferred_element_type=jnp.float32)
         + jnp.dot(vv, B, preferred_element_type=jnp.float32)
         + jnp.dot(jnp.abs(uu - vv), C, preferred_element_type=jnp.float32)
         + jnp.dot(uu * vv, Dm, preferred_element_type=jnp.float32)
         + mb1[...])
    hid = jnp.maximum(t, 0.0)
    out[...] = jnp.dot(hid, mW2[...], preferred_element_type=jnp.float32) + mb2[...]


def _row_spec(r, cols):
    return pl.BlockSpec((r, cols), lambda i: (i, 0))


def _full_spec(shape):
    nd = len(shape)
    return pl.BlockSpec(shape, lambda i: (0,) * nd)


def kernel(edge_index, pairs, emb, W1, b1, W2, b2, mW1, mb1, mW2, mb2):
    f32 = jnp.float32
    src = edge_index[0]
    dst = edge_index[1]
    pad = EP - E
    pad_iota = jnp.arange(pad, dtype=jnp.int32)
    src_p = jnp.concatenate([src, pad_iota % N])
    dst_p = jnp.concatenate([dst, N + pad_iota % (NP - N)])
    srch = jnp.concatenate(
        [jnp.stack([src_p * 2, src_p * 2 + 1]).reshape(2, EP // 128, 128),
         jnp.zeros((2, IR, 128), jnp.int32)], axis=1)
    dst2d = jnp.concatenate(
        [dst_p.reshape(EP // 128, 128), jnp.zeros((IR, 128), jnp.int32)])
    pidx = pairs.T.reshape(2 * P)
    b1r = b1.reshape(1, D)
    b2r = b2.reshape(1, D)
    mb1r = mb1.reshape(1, D)
    mb2r = mb2.reshape(1, 1)

    degp = _deg_kernel(dst2d)
    d0 = degp[:N].reshape(N, 1)
    d1 = degp[NP:NP + N].reshape(N, 1)

    grid_a = (N // _RA,)
    y1, s1, dis, inv = pl.pallas_call(
        _tc_a_body,
        grid=grid_a,
        in_specs=[_row_spec(_RA, D), _full_spec((D, D)), _full_spec((1, D)),
                  _row_spec(_RA, 1), _row_spec(_RA, 1)],
        out_specs=[_row_spec(_RA, D), _row_spec(_RA, D),
                   _row_spec(_RA, 1), _row_spec(_RA, 1)],
        out_shape=[jax.ShapeDtypeStruct((N, D), f32),
                   jax.ShapeDtypeStruct((N, D), f32),
                   jax.ShapeDtypeStruct((N, 1), f32),
                   jax.ShapeDtypeStruct((N, 1), f32)],
    )(emb, W1, b1r, d0, d1)

    y1r = y1.reshape(2 * N, 32)
    g = _agg_kernel(y1r, srch, dst2d)
    g0, g1 = g[0, :N], g[1, :N]

    y2, s2 = pl.pallas_call(
        _tc_b_body,
        grid=grid_a,
        in_specs=[_row_spec(_RA, 32)] * 2
        + [_row_spec(_RA, D), _row_spec(_RA, 1), _row_spec(_RA, 1),
           _full_spec((D, D)), _full_spec((1, D))],
        out_specs=[_row_spec(_RA, D), _row_spec(_RA, D)],
        out_shape=[jax.ShapeDtypeStruct((N, D), f32),
                   jax.ShapeDtypeStruct((N, D), f32)],
    )(g0, g1, s1, dis, inv, W2, b2r)

    y2r = y2.reshape(2 * N, 32)
    g = _agg_kernel(y2r, srch, dst2d)
    g0, g1 = g[0, :N], g[1, :N]

    h = pl.pallas_call(
        _tc_c_body,
        grid=grid_a,
        in_specs=[_row_spec(_RA, 32)] * 2 + [_row_spec(_RA, D), _row_spec(_RA, 1)],
        out_specs=_row_spec(_RA, D),
        out_shape=jax.ShapeDtypeStruct((N, D), f32),
    )(g0, g1, s2, dis)

    u, v = _pair_gather_kernel(h, pidx)

    logits = pl.pallas_call(
        _tc_d_body,
        grid=(P // _RD,),
        in_specs=[_row_spec(_RD, D), _row_spec(_RD, D),
                  _full_spec((4 * D, D)), _full_spec((1, D)),
                  _full_spec((D, 1)), _full_spec((1, 1))],
        out_specs=_row_spec(_RD, 1),
        out_shape=jax.ShapeDtypeStruct((P, 1), f32),
    )(u, v, mW1, mb1r, mW2, mb2r)
    return logits.reshape(P)


# speedup vs baseline: 24.9874x; 1.0117x over previous
import functools

import jax
import jax.numpy as jnp
from jax import lax
from jax.experimental import pallas as pl
from jax.experimental.pallas import tpu as pltpu
from jax.experimental.pallas import tpu_sc as plsc

N = 50000
D = 64
E = 800000
P = 65536

NC = 2
NS = 16
NW = NC * NS
L = 16

CK = 512
EW = 25600
EP = NW * EW
NCHUNK = EW // CK
NP = 50176
SLAB = NP // NS
IR = CK // 128

_mesh = plsc.VectorSubcoreMesh(core_axis_name="c", subcore_axis_name="s")


def _wid():
    return lax.axis_index("c") * NS + lax.axis_index("s")


@functools.partial(
    pl.kernel,
    out_type=jax.ShapeDtypeStruct((NC * NP,), jnp.float32),
    mesh=_mesh,
    compiler_params=pltpu.CompilerParams(use_tc_tiling_on_sc=False),
    scratch_types=[
        pltpu.VMEM((2, IR, 128), jnp.int32),
        pltpu.VMEM((128,), jnp.float32),
        pltpu.VMEM((SLAB,), jnp.float32),
        pltpu.VMEM_SHARED((NP,), jnp.float32),
        pltpu.SemaphoreType.DMA,
        pltpu.SemaphoreType.DMA,
    ],
)
def _deg_kernel(dst2d_hbm, out_hbm, didx_v, ones_v, buf_v, acc_sh, ssem, wsem):
    cid = lax.axis_index("c")
    sid = lax.axis_index("s")
    wid = cid * NS + sid

    zero16 = jnp.zeros((L,), jnp.float32)
    one16 = jnp.ones((L,), jnp.float32)
    for q in range(128 // L):
        ones_v[pl.ds(q * L, L)] = one16

    def zb(i, carry):
        buf_v[pl.ds(i * L, L)] = zero16
        return carry

    lax.fori_loop(0, SLAB // L, zb, 0)
    pltpu.sync_copy(buf_v, acc_sh.at[pl.ds(sid * SLAB, SLAB)])
    plsc.subcore_barrier()

    def _rows(c):
        return pl.multiple_of(wid * (EW // 128) + c * IR, IR)

    def _issue(c, par):
        pltpu.async_copy(dst2d_hbm.at[pl.ds(_rows(c), IR)], didx_v.at[par], wsem)

    def _wait(c, par):
        pltpu.make_async_copy(dst2d_hbm.at[pl.ds(_rows(c), IR)],
                              didx_v.at[par], wsem).wait()

    def _scat(par):
        descs = [
            pltpu.async_copy(ones_v, acc_sh.at[didx_v.at[par, j]], ssem, add=True)
            for j in range(IR)
        ]
        for d in descs:
            d.wait()

    _issue(0, 0)

    def chunk2(i, carry):
        for sub in range(2):
            c = i * 2 + sub
            _wait(c, sub)
            _issue(c + 1, 1 - sub)
            _scat(sub)
        return carry

    lax.fori_loop(0, NCHUNK // 2, chunk2, 0)
    _wait(NCHUNK, 0)
    plsc.subcore_barrier()
    pltpu.sync_copy(acc_sh.at[pl.ds(sid * SLAB, SLAB)], buf_v)
    pltpu.sync_copy(buf_v, out_hbm.at[pl.ds(cid * NP + sid * SLAB, SLAB)])


EW2 = EP // NS
NCHUNK2 = EW2 // CK


@functools.partial(
    pl.kernel,
    out_type=jax.ShapeDtypeStruct((NC, NP, 32), jnp.float32),
    mesh=_mesh,
    compiler_params=pltpu.CompilerParams(use_tc_tiling_on_sc=False),
    scratch_types=[
        pltpu.VMEM((2, IR, 128), jnp.int32),
        pltpu.VMEM((2, IR, 128), jnp.int32),
        pltpu.VMEM((CK, 32), jnp.float32),
        pltpu.VMEM_SHARED((NP, 32), jnp.float32),
        pltpu.SemaphoreType.DMA,
        pltpu.SemaphoreType.DMA,
        pltpu.SemaphoreType.DMA,
    ],
)
def _agg_kernel(y2_hbm, srch_hbm, dst2d_hbm, out_hbm,
                sidx_v, didx_v, rows_v, acc_sh, gsem, ssem, wsem):
    cid = lax.axis_index("c")
    sid = lax.axis_index("s")

    zero16 = jnp.zeros((L,), jnp.float32)

    def zb(r, carry):
        rows_v[r, pl.ds(0, L)] = zero16
        rows_v[r, pl.ds(L, L)] = zero16
        return carry

    lax.fori_loop(0, CK, zb, 0)
    pieces = [(o, CK) for o in range(0, (SLAB // CK) * CK, CK)]
    if SLAB % CK:
        pieces.append(((SLAB // CK) * CK, SLAB % CK))
    for off, sz in pieces:
        pltpu.sync_copy(rows_v.at[pl.ds(0, sz)],
                        acc_sh.at[pl.ds(sid * SLAB + off, sz)])
    plsc.subcore_barrier()

    def _idx_rows(c):
        return pl.multiple_of(sid * (EW2 // 128) + c * IR, IR)

    def _issue_idx(c, par):
        r = _idx_rows(c)
        pltpu.async_copy(srch_hbm.at[cid, pl.ds(r, IR)], sidx_v.at[par], wsem)
        pltpu.async_copy(dst2d_hbm.at[pl.ds(r, IR)], didx_v.at[par], wsem)

    def _wait_idx(c, par):
        r = _idx_rows(c)
        pltpu.make_async_copy(srch_hbm.at[cid, pl.ds(r, IR)],
                              sidx_v.at[par], wsem).wait()
        pltpu.make_async_copy(dst2d_hbm.at[pl.ds(r, IR)],
                              didx_v.at[par], wsem).wait()

    _issue_idx(0, 0)

    def chunk2(i, carry):
        for sub in range(2):
            c = i * 2 + sub
            par = sub
            _wait_idx(c, par)
            _issue_idx(c + 1, 1 - par)
            gd = [
                pltpu.async_copy(y2_hbm.at[sidx_v.at[par, j]],
                                 rows_v.at[pl.ds(j * 128, 128)], gsem)
                for j in range(IR)
            ]
            sd = []
            for j in range(IR):
                gd[j].wait()
                sd.append(pltpu.async_copy(
                    rows_v.at[pl.ds(j * 128, 128)],
                    acc_sh.at[didx_v.at[par, j]], ssem, add=True))
            for d in sd:
                d.wait()
        return carry

    lax.fori_loop(0, NCHUNK2 // 2, chunk2, 0)
    _wait_idx(NCHUNK2, 0)
    plsc.subcore_barrier()
    for off, sz in pieces:
        pltpu.sync_copy(acc_sh.at[pl.ds(sid * SLAB + off, sz)],
                        rows_v.at[pl.ds(0, sz)])
        pltpu.sync_copy(rows_v.at[pl.ds(0, sz)],
                        out_hbm.at[cid, pl.ds(sid * SLAB + off, sz)])


_PCHUNK = P // NW // 128


@functools.partial(
    pl.kernel,
    out_type=(jax.ShapeDtypeStruct((P, D), jnp.float32),
              jax.ShapeDtypeStruct((P, D), jnp.float32)),
    mesh=_mesh,
    compiler_params=pltpu.CompilerParams(use_tc_tiling_on_sc=False),
    scratch_types=[
        pltpu.VMEM((2, 128), jnp.int32),
        pltpu.VMEM((2, 128), jnp.int32),
        pltpu.VMEM((128, D), jnp.float32),
        pltpu.VMEM((128, D), jnp.float32),
        pltpu.SemaphoreType.DMA,
        pltpu.SemaphoreType.DMA,
    ],
)
def _pair_gather_kernel(h_hbm, pidx_hbm, u_hbm, v_hbm,
                        uidx_v, vidx_v, ubuf_v, vbuf_v, gsem, wsem):
    wid = _wid()

    def _off(c):
        return pl.multiple_of((wid * _PCHUNK + c) * 128, 8)

    def _issue_idx(c, par):
        o = _off(c)
        pltpu.async_copy(pidx_hbm.at[pl.ds(o, 128)], uidx_v.at[par], wsem)
        pltpu.async_copy(pidx_hbm.at[pl.ds(P + o, 128)], vidx_v.at[par], wsem)

    def _wait_idx(c, par):
        o = _off(c)
        pltpu.make_async_copy(pidx_hbm.at[pl.ds(o, 128)],
                              uidx_v.at[par], wsem).wait()
        pltpu.make_async_copy(pidx_hbm.at[pl.ds(P + o, 128)],
                              vidx_v.at[par], wsem).wait()

    _issue_idx(0, 0)

    def chunk2(i, carry):
        for sub in range(2):
            c = i * 2 + sub
            _wait_idx(c, sub)
            _issue_idx(c + 1, 1 - sub)
            o = _off(c)
            gu = pltpu.async_copy(h_hbm.at[uidx_v.at[sub]], ubuf_v, gsem)
            gv = pltpu.async_copy(h_hbm.at[vidx_v.at[sub]], vbuf_v, gsem)
            gu.wait()
            gv.wait()
            wu = pltpu.async_copy(ubuf_v, u_hbm.at[pl.ds(o, 128)], wsem)
            wv = pltpu.async_copy(vbuf_v, v_hbm.at[pl.ds(o, 128)], wsem)
            wu.wait()
            wv.wait()
        return carry

    lax.fori_loop(0, _PCHUNK // 2, chunk2, 0)
    _wait_idx(_PCHUNK, 0)


_RA = 2000


def _tc_a_body(emb, W1, b1, d0, d1, y1, s1, dis, inv):
    deg = d0[...] + d1[...] + 1.0
    di = lax.rsqrt(deg)
    iv = 1.0 / deg
    xw = jnp.dot(emb[...], W1[...], preferred_element_type=jnp.float32)
    y1[...] = xw * di
    s1[...] = xw * iv + b1[...]
    dis[...] = di
    inv[...] = iv


def _tc_b_body(g0, g1, s1, dis, inv, W2, b2, y2, s2):
    agg = jnp.concatenate([g0[...], g1[...]], axis=1)
    h1 = jnp.maximum(dis[...] * agg + s1[...], 0.0)
    xw = jnp.dot(h1, W2[...], preferred_element_type=jnp.float32)
    y2[...] = xw * dis[...]
    s2[...] = xw * inv[...] + b2[...]


def _tc_c_body(g0, g1, s2, dis, h):
    agg = jnp.concatenate([g0[...], g1[...]], axis=1)
    h[...] = dis[...] * agg + s2[...]


_RD = 4096


def _tc_d_body(u, v, mW1, mb1, mW2, mb2, out):
    uu = u[...]
    vv = v[...]
    A = mW1[0:D, :]
    B = mW1[D:2 * D, :]
    C = mW1[2 * D:3 * D, :]
    Dm = mW1[3 * D:4 * D, :]
    t = (jnp.dot(uu, A, preferred_element_type=jnp.float32)
         + jnp.dot(vv, B, preferred_element_type=jnp.float32)
         + jnp.dot(jnp.abs(uu - vv), C, preferred_element_type=jnp.float32)
         + jnp.dot(uu * vv, Dm, preferred_element_type=jnp.float32)
         + mb1[...])
    hid = jnp.maximum(t, 0.0)
    out[...] = jnp.dot(hid, mW2[...], preferred_element_type=jnp.float32) + mb2[...]


def _row_spec(r, cols):
    return pl.BlockSpec((r, cols), lambda i: (i, 0))


def _full_spec(shape):
    nd = len(shape)
    return pl.BlockSpec(shape, lambda i: (0,) * nd)


def kernel(edge_index, pairs, emb, W1, b1, W2, b2, mW1, mb1, mW2, mb2):
    f32 = jnp.float32
    src = edge_index[0]
    dst = edge_index[1]
    pad = EP - E
    pad_iota = jnp.arange(pad, dtype=jnp.int32)
    src_p = jnp.concatenate([src, pad_iota % N])
    dst_p = jnp.concatenate([dst, N + pad_iota % (NP - N)])
    srch = jnp.concatenate(
        [jnp.stack([src_p * 2, src_p * 2 + 1]).reshape(2, EP // 128, 128),
         jnp.zeros((2, IR, 128), jnp.int32)], axis=1)
    dst2d = jnp.concatenate(
        [dst_p.reshape(EP // 128, 128), jnp.zeros((IR, 128), jnp.int32)])
    pidx = jnp.concatenate([pairs.T.reshape(2 * P),
                            jnp.zeros((128,), jnp.int32)])
    b1r = b1.reshape(1, D)
    b2r = b2.reshape(1, D)
    mb1r = mb1.reshape(1, D)
    mb2r = mb2.reshape(1, 1)

    degp = _deg_kernel(dst2d)
    d0 = degp[:N].reshape(N, 1)
    d1 = degp[NP:NP + N].reshape(N, 1)

    grid_a = (N // _RA,)
    y1, s1, dis, inv = pl.pallas_call(
        _tc_a_body,
        grid=grid_a,
        in_specs=[_row_spec(_RA, D), _full_spec((D, D)), _full_spec((1, D)),
                  _row_spec(_RA, 1), _row_spec(_RA, 1)],
        out_specs=[_row_spec(_RA, D), _row_spec(_RA, D),
                   _row_spec(_RA, 1), _row_spec(_RA, 1)],
        out_shape=[jax.ShapeDtypeStruct((N, D), f32),
                   jax.ShapeDtypeStruct((N, D), f32),
                   jax.ShapeDtypeStruct((N, 1), f32),
                   jax.ShapeDtypeStruct((N, 1), f32)],
    )(emb, W1, b1r, d0, d1)

    y1r = y1.reshape(2 * N, 32)
    g = _agg_kernel(y1r, srch, dst2d)
    g0, g1 = g[0, :N], g[1, :N]

    y2, s2 = pl.pallas_call(
        _tc_b_body,
        grid=grid_a,
        in_specs=[_row_spec(_RA, 32)] * 2
        + [_row_spec(_RA, D), _row_spec(_RA, 1), _row_spec(_RA, 1),
           _full_spec((D, D)), _full_spec((1, D))],
        out_specs=[_row_spec(_RA, D), _row_spec(_RA, D)],
        out_shape=[jax.ShapeDtypeStruct((N, D), f32),
                   jax.ShapeDtypeStruct((N, D), f32)],
    )(g0, g1, s1, dis, inv, W2, b2r)

    y2r = y2.reshape(2 * N, 32)
    g = _agg_kernel(y2r, srch, dst2d)
    g0, g1 = g[0, :N], g[1, :N]

    h = pl.pallas_call(
        _tc_c_body,
        grid=grid_a,
        in_specs=[_row_spec(_RA, 32)] * 2 + [_row_spec(_RA, D), _row_spec(_RA, 1)],
        out_specs=_row_spec(_RA, D),
        out_shape=jax.ShapeDtypeStruct((N, D), f32),
    )(g0, g1, s2, dis)

    u, v = _pair_gather_kernel(h, pidx)

    logits = pl.pallas_call(
        _tc_d_body,
        grid=(P // _RD,),
        in_specs=[_row_spec(_RD, D), _row_spec(_RD, D),
                  _full_spec((4 * D, D)), _full_spec((1, D)),
                  _full_spec((D, 1)), _full_spec((1, 1))],
        out_specs=_row_spec(_RD, 1),
        out_shape=jax.ShapeDtypeStruct((P, 1), f32),
    )(u, v, mW1, mb1r, mW2, mb2r)
    return logits.reshape(P)
